# baseline (speedup 1.0000x reference)
import functools

import jax
import jax.numpy as jnp
from jax import lax
from jax.experimental import pallas as pl
from jax.experimental.pallas import tpu as pltpu
from jax.experimental.pallas import tpu_sc as plsc

BATCH = 16384
DIM = 100
ROW_PAD = 104

_INFO = plsc.get_sparse_core_info()
_NC = _INFO.num_cores
_NS = _INFO.num_subcores
NW = _NC * _NS
LANES = 16

VOCAB_PAD = 1000064
N_TILE_COLS = VOCAB_PAD // 128
STRIPS_BASE = N_TILE_COLS // NW
STRIPS_REM = N_TILE_COLS % NW

HIT_CAP = 768
N_IDX_CHUNKS = BATCH // LANES
SLOTS = 16
SLOT_STRIDE = 112
NBUCKET = 272


def _splat(x):
    return jnp.full((LANES,), x, jnp.int32)


@functools.partial(
    pl.kernel,
    mesh=plsc.VectorSubcoreMesh(core_axis_name="c", subcore_axis_name="s"),
    out_type=jax.ShapeDtypeStruct((BATCH * ROW_PAD,), jnp.float32),
    scratch_types=[
        pltpu.VMEM((BATCH,), jnp.int32),
        pltpu.VMEM((100, 128), jnp.float32),
        pltpu.VMEM((100, 128), jnp.float32),
        pltpu.VMEM((HIT_CAP + LANES,), jnp.int32),
        pltpu.VMEM((HIT_CAP + LANES,), jnp.int32),
        pltpu.VMEM((HIT_CAP + LANES,), jnp.int32),
        pltpu.VMEM((HIT_CAP + LANES,), jnp.int32),
        pltpu.VMEM((NBUCKET,), jnp.int32),
        pltpu.VMEM((NBUCKET,), jnp.int32),
        pltpu.VMEM((NBUCKET,), jnp.int32),
        pltpu.VMEM((SLOTS * SLOT_STRIDE + LANES,), jnp.float32),
        pltpu.SemaphoreType.DMA,
        pltpu.SemaphoreType.DMA,
    ],
    compiler_params=pltpu.CompilerParams(
        use_tc_tiling_on_sc=True, needs_layout_passes=False),
)
def _emb_stream(idx_hbm, table_t_hbm, out_hbm, idx_all, strip_a, strip_b,
                hit_r, hit_i, sr_s, si_s, hist, bases, nxt, outbuf,
                ssem, osem):
    wid = lax.axis_index("s") * _NC + lax.axis_index("c")
    lo_strip = wid * STRIPS_BASE + jnp.minimum(wid, STRIPS_REM)
    n_strips = STRIPS_BASE + jnp.where(wid < STRIPS_REM, 1, 0)
    lo_c = lo_strip
    hi_c = lo_strip + n_strips

    pltpu.sync_copy(idx_hbm, idx_all)

    iota = lax.iota(jnp.int32, LANES)
    ones = jnp.full((LANES,), 1, jnp.int32)
    zeros = jnp.full((LANES,), 0, jnp.int32)
    lane0 = iota == zeros

    def scan(n, nh):
        v = idx_all[pl.ds(n * LANES, LANES)]
        c = lax.shift_right_logical(v, 7)
        m = (c >= _splat(lo_c)) & (c < _splat(hi_c))
        cnt = plsc.cumsum(jnp.where(m, ones, zeros))
        dest = _splat(nh) + cnt - 1
        plsc.store_scatter(hit_r, [dest], v, mask=m)
        pos = _splat(n * LANES) + iota
        plsc.store_scatter(hit_i, [dest], pos, mask=m)
        return nh + cnt[LANES - 1]
    nh = lax.fori_loop(0, N_IDX_CHUNKS, scan, jnp.int32(0), unroll=False)

    def clear(n, _):
        hist[pl.ds(n * LANES, LANES)] = zeros
        return ()
    lax.fori_loop(0, NBUCKET // LANES, clear, (), unroll=False)

    def count(j, _):
        rv = hit_r[pl.ds(j, LANES)]
        strip = lax.shift_right_logical(rv[0], 7) - lo_strip
        ch = lax.shift_right_logical(strip, 4)
        lp = lax.rem(strip, jnp.int32(LANES))
        hv = hist[pl.ds(ch * LANES, LANES)]
        hist[pl.ds(ch * LANES, LANES)] = hv + jnp.where(
            iota == _splat(lp), ones, zeros)
        return ()
    lax.fori_loop(0, nh, count, (), unroll=False)

    def prefix(n, carry):
        hv = hist[pl.ds(n * LANES, LANES)]
        cs = plsc.cumsum(hv)
        bases[pl.ds(n * LANES, LANES)] = _splat(carry) + cs - hv
        nxt[pl.ds(n * LANES, LANES)] = _splat(carry) + cs - hv
        return carry + cs[LANES - 1]
    lax.fori_loop(0, NBUCKET // LANES, prefix, jnp.int32(0), unroll=False)

    def place(j, _):
        rv = hit_r[pl.ds(j, LANES)]
        iv = hit_i[pl.ds(j, LANES)]
        r = rv[0]
        i_out = iv[0]
        strip = lax.shift_right_logical(r, 7) - lo_strip
        sv = _splat(strip)
        n0 = plsc.load_gather(nxt, [sv])[0]
        nv0 = _splat(n0)
        plsc.store_scatter(sr_s, [nv0], _splat(r), mask=lane0)
        plsc.store_scatter(si_s, [nv0], _splat(i_out), mask=lane0)
        plsc.store_scatter(nxt, [sv], nv0 + 1, mask=lane0)
        return ()
    lax.fori_loop(0, nh, place, (), unroll=False)

    def fetch_strip(g, buf):
        col0 = pl.multiple_of((lo_strip + g) * 128, 128)
        return pltpu.async_copy(table_t_hbm.at[:, pl.ds(col0, 128)], buf, ssem)

    fetch_strip(0, strip_a)

    def extract_hits(k0, k1, buf):
        def one(k, _):
            rv = sr_s[pl.ds(k, LANES)]
            iv = si_s[pl.ds(k, LANES)]
            r = rv[0]
            i_out = iv[0]
            lane = lax.rem(r, jnp.int32(128))
            colv = _splat(lane)
            slot = lax.rem(k, jnp.int32(SLOTS))
            sbase = slot * SLOT_STRIDE
            for t in range(7):
                rowv = iota + t * LANES
                if t == 6:
                    rowv = jnp.minimum(rowv, DIM - 1)
                g16 = plsc.load_gather(buf, [rowv, colv])
                outbuf[pl.ds(sbase + t * LANES, LANES)] = g16

            @pl.when(k >= SLOTS)
            def _():
                pltpu.make_async_copy(
                    out_hbm.at[pl.ds(0, ROW_PAD)],
                    outbuf.at[pl.ds(sbase, ROW_PAD)],
                    osem,
                ).wait()

            pltpu.async_copy(
                outbuf.at[pl.ds(sbase, ROW_PAD)],
                out_hbm.at[pl.ds(i_out * ROW_PAD, ROW_PAD)],
                osem,
            )
            return ()
        lax.fori_loop(k0, k1, one, (), unroll=False)

    def strip_body(g, _):
        gv = _splat(g)
        start = plsc.load_gather(bases, [gv])[0]
        cnt_g = plsc.load_gather(hist, [gv])[0]
        end = start + cnt_g

        parity = lax.rem(g, jnp.int32(2))

        def wait_strip(buf):
            pltpu.make_async_copy(
                table_t_hbm.at[:, pl.ds(0, 128)], buf, ssem).wait()

        @pl.when(parity == 0)
        def _():
            wait_strip(strip_a)

            @pl.when(g + 1 < n_strips)
            def _():
                fetch_strip(g + 1, strip_b)

            extract_hits(start, end, strip_a)

        @pl.when(parity == 1)
        def _():
            wait_strip(strip_b)

            @pl.when(g + 1 < n_strips)
            def _():
                fetch_strip(g + 1, strip_a)

            extract_hits(start, end, strip_b)

        return ()

    lax.fori_loop(0, n_strips, strip_body, (), unroll=False)

    def final_drain(_, __):
        pltpu.make_async_copy(
            out_hbm.at[pl.ds(0, ROW_PAD)],
            outbuf.at[pl.ds(0, ROW_PAD)],
            osem,
        ).wait()
        return ()
    lax.fori_loop(0, jnp.minimum(nh, SLOTS), final_drain, (), unroll=False)


def kernel(inputs, table):
    idx = inputs.reshape(BATCH)
    out_pad = _emb_stream(idx, table.T)
    return out_pad.reshape(BATCH, ROW_PAD)[:, :DIM]


# speedup vs baseline: 1.6454x; 1.6454x over previous
import functools

import jax
import jax.numpy as jnp
from jax import lax
from jax.experimental import pallas as pl
from jax.experimental.pallas import tpu as pltpu
from jax.experimental.pallas import tpu_sc as plsc

BATCH = 16384
DIM = 100
ROW_PAD = 104

_INFO = plsc.get_sparse_core_info()
_NC = _INFO.num_cores
_NS = _INFO.num_subcores
NW = _NC * _NS
LANES = 16

VOCAB_PAD = 1000064
N_TILE_COLS = VOCAB_PAD // 128
STRIPS_BASE = N_TILE_COLS // NW
STRIPS_REM = N_TILE_COLS % NW

HIT_CAP = 768
N_IDX_CHUNKS = BATCH // LANES
SLOTS = 16
SLOT_STRIDE = 112
NBUCKET = 272


def _splat(x):
    return jnp.full((LANES,), x, jnp.int32)


@functools.partial(
    pl.kernel,
    mesh=plsc.VectorSubcoreMesh(core_axis_name="c", subcore_axis_name="s"),
    out_type=jax.ShapeDtypeStruct((BATCH * ROW_PAD,), jnp.float32),
    scratch_types=[
        pltpu.VMEM((BATCH,), jnp.int32),
        pltpu.VMEM((100, 128), jnp.float32),
        pltpu.VMEM((100, 128), jnp.float32),
        pltpu.VMEM((100, 128), jnp.float32),
        pltpu.VMEM((100, 128), jnp.float32),
        pltpu.VMEM((HIT_CAP + LANES,), jnp.int32),
        pltpu.VMEM((HIT_CAP + LANES,), jnp.int32),
        pltpu.VMEM((HIT_CAP + LANES,), jnp.int32),
        pltpu.VMEM((HIT_CAP + LANES,), jnp.int32),
        pltpu.VMEM((NBUCKET,), jnp.int32),
        pltpu.VMEM((NBUCKET,), jnp.int32),
        pltpu.VMEM((NBUCKET,), jnp.int32),
        pltpu.VMEM((SLOTS * SLOT_STRIDE + LANES,), jnp.float32),
        pltpu.SemaphoreType.DMA,
        pltpu.SemaphoreType.DMA,
        pltpu.SemaphoreType.DMA,
        pltpu.SemaphoreType.DMA,
        pltpu.SemaphoreType.DMA,
    ],
    compiler_params=pltpu.CompilerParams(
        use_tc_tiling_on_sc=True, needs_layout_passes=False),
)
def _emb_stream(idx_hbm, table_t_hbm, out_hbm, idx_all, strip_0, strip_1,
                strip_2, strip_3, hit_r, hit_i, sr_s, si_s, hist, bases,
                nxt, outbuf, ssem_0, ssem_1, ssem_2, ssem_3, osem):
    wid = lax.axis_index("s") * _NC + lax.axis_index("c")
    lo_strip = wid * STRIPS_BASE + jnp.minimum(wid, STRIPS_REM)
    n_strips = STRIPS_BASE + jnp.where(wid < STRIPS_REM, 1, 0)
    lo_c = lo_strip
    hi_c = lo_strip + n_strips

    pltpu.sync_copy(idx_hbm, idx_all)

    strips = (strip_0, strip_1, strip_2, strip_3)
    ssems = (ssem_0, ssem_1, ssem_2, ssem_3)
    NBUF = 4

    def fetch_strip(g, m):
        col0 = pl.multiple_of((lo_strip + g) * 128, 128)
        return pltpu.async_copy(
            table_t_hbm.at[:, pl.ds(col0, 128)], strips[m], ssems[m])

    for m in range(NBUF):
        fetch_strip(m, m)

    iota = lax.iota(jnp.int32, LANES)
    ones = jnp.full((LANES,), 1, jnp.int32)
    zeros = jnp.full((LANES,), 0, jnp.int32)
    lane0 = iota == zeros

    def scan(n, nh):
        v = idx_all[pl.ds(n * LANES, LANES)]
        c = lax.shift_right_logical(v, 7)
        m = (c >= _splat(lo_c)) & (c < _splat(hi_c))
        cnt = plsc.cumsum(jnp.where(m, ones, zeros))
        dest = _splat(nh) + cnt - 1
        plsc.store_scatter(hit_r, [dest], v, mask=m)
        pos = _splat(n * LANES) + iota
        plsc.store_scatter(hit_i, [dest], pos, mask=m)
        return nh + cnt[LANES - 1]
    nh = lax.fori_loop(0, N_IDX_CHUNKS, scan, jnp.int32(0), unroll=False)

    def clear(n, _):
        hist[pl.ds(n * LANES, LANES)] = zeros
        return ()
    lax.fori_loop(0, NBUCKET // LANES, clear, (), unroll=False)

    def count(j, _):
        rv = hit_r[pl.ds(j, LANES)]
        strip = lax.shift_right_logical(rv[0], 7) - lo_strip
        ch = lax.shift_right_logical(strip, 4)
        lp = lax.rem(strip, jnp.int32(LANES))
        hv = hist[pl.ds(ch * LANES, LANES)]
        hist[pl.ds(ch * LANES, LANES)] = hv + jnp.where(
            iota == _splat(lp), ones, zeros)
        return ()
    lax.fori_loop(0, nh, count, (), unroll=False)

    def prefix(n, carry):
        hv = hist[pl.ds(n * LANES, LANES)]
        cs = plsc.cumsum(hv)
        bases[pl.ds(n * LANES, LANES)] = _splat(carry) + cs - hv
        nxt[pl.ds(n * LANES, LANES)] = _splat(carry) + cs - hv
        return carry + cs[LANES - 1]
    lax.fori_loop(0, NBUCKET // LANES, prefix, jnp.int32(0), unroll=False)

    def place(j, _):
        rv = hit_r[pl.ds(j, LANES)]
        iv = hit_i[pl.ds(j, LANES)]
        r = rv[0]
        i_out = iv[0]
        strip = lax.shift_right_logical(r, 7) - lo_strip
        sv = _splat(strip)
        n0 = plsc.load_gather(nxt, [sv])[0]
        nv0 = _splat(n0)
        plsc.store_scatter(sr_s, [nv0], _splat(r), mask=lane0)
        plsc.store_scatter(si_s, [nv0], _splat(i_out), mask=lane0)
        plsc.store_scatter(nxt, [sv], nv0 + 1, mask=lane0)
        return ()
    lax.fori_loop(0, nh, place, (), unroll=False)

    def extract_hits(k0, k1, buf):
        def one(k, _):
            rv = sr_s[pl.ds(k, LANES)]
            iv = si_s[pl.ds(k, LANES)]
            r = rv[0]
            i_out = iv[0]
            lane = lax.rem(r, jnp.int32(128))
            colv = _splat(lane)
            slot = lax.rem(k, jnp.int32(SLOTS))
            sbase = slot * SLOT_STRIDE
            for t in range(7):
                rowv = iota + t * LANES
                if t == 6:
                    rowv = jnp.minimum(rowv, DIM - 1)
                g16 = plsc.load_gather(buf, [rowv, colv])
                outbuf[pl.ds(sbase + t * LANES, LANES)] = g16

            @pl.when(k >= SLOTS)
            def _():
                pltpu.make_async_copy(
                    out_hbm.at[pl.ds(0, ROW_PAD)],
                    outbuf.at[pl.ds(sbase, ROW_PAD)],
                    osem,
                ).wait()

            pltpu.async_copy(
                outbuf.at[pl.ds(sbase, ROW_PAD)],
                out_hbm.at[pl.ds(i_out * ROW_PAD, ROW_PAD)],
                osem,
            )
            return ()
        lax.fori_loop(k0, k1, one, (), unroll=False)

    def strip_body(g, _):
        gv = _splat(g)
        start = plsc.load_gather(bases, [gv])[0]
        cnt_g = plsc.load_gather(hist, [gv])[0]
        end = start + cnt_g

        mod = lax.rem(g, jnp.int32(NBUF))
        for m in range(NBUF):
            @pl.when(mod == m)
            def _(m=m):
                pltpu.make_async_copy(
                    table_t_hbm.at[:, pl.ds(0, 128)],
                    strips[m], ssems[m]).wait()
                extract_hits(start, end, strips[m])

                @pl.when(g + NBUF < n_strips)
                def _():
                    fetch_strip(g + NBUF, m)

        return ()

    lax.fori_loop(0, n_strips, strip_body, (), unroll=False)

    def final_drain(_, __):
        pltpu.make_async_copy(
            out_hbm.at[pl.ds(0, ROW_PAD)],
            outbuf.at[pl.ds(0, ROW_PAD)],
            osem,
        ).wait()
        return ()
    lax.fori_loop(0, jnp.minimum(nh, SLOTS), final_drain, (), unroll=False)


def kernel(inputs, table):
    idx = inputs.reshape(BATCH)
    out_pad = _emb_stream(idx, table.T)
    return out_pad.reshape(BATCH, ROW_PAD)[:, :DIM]


# speedup vs baseline: 1.7456x; 1.0609x over previous
import functools

import jax
import jax.numpy as jnp
from jax import lax
from jax.experimental import pallas as pl
from jax.experimental.pallas import tpu as pltpu
from jax.experimental.pallas import tpu_sc as plsc

BATCH = 16384
DIM = 100
ROW_PAD = 104

_INFO = plsc.get_sparse_core_info()
_NC = _INFO.num_cores
_NS = _INFO.num_subcores
NW = _NC * _NS
LANES = 16

VOCAB_PAD = 1000064
N_TILE_COLS = VOCAB_PAD // 128
STRIPS_BASE = N_TILE_COLS // NW
STRIPS_REM = N_TILE_COLS % NW

HIT_CAP = 1024
N_IDX_CHUNKS = BATCH // LANES
SLOTS = 16
SLOT_STRIDE = 112
NBUCKET = 272


def _splat(x):
    return jnp.full((LANES,), x, jnp.int32)


@functools.partial(
    pl.kernel,
    mesh=plsc.VectorSubcoreMesh(core_axis_name="c", subcore_axis_name="s"),
    out_type=jax.ShapeDtypeStruct((BATCH * ROW_PAD,), jnp.float32),
    scratch_types=[
        pltpu.VMEM((BATCH,), jnp.int32),
        pltpu.VMEM((100, 128), jnp.float32),
        pltpu.VMEM((100, 128), jnp.float32),
        pltpu.VMEM((100, 128), jnp.float32),
        pltpu.VMEM((100, 128), jnp.float32),
        pltpu.VMEM((100, 128), jnp.float32),
        pltpu.VMEM((100, 128), jnp.float32),
        pltpu.VMEM((HIT_CAP + LANES,), jnp.int32),
        pltpu.VMEM((HIT_CAP + LANES,), jnp.int32),
        pltpu.VMEM((HIT_CAP + LANES,), jnp.int32),
        pltpu.VMEM((HIT_CAP + LANES,), jnp.int32),
        pltpu.VMEM((NBUCKET,), jnp.int32),
        pltpu.VMEM((NBUCKET,), jnp.int32),
        pltpu.VMEM((NBUCKET,), jnp.int32),
        pltpu.VMEM((SLOTS * SLOT_STRIDE + LANES,), jnp.float32),
        pltpu.SemaphoreType.DMA,
        pltpu.SemaphoreType.DMA,
        pltpu.SemaphoreType.DMA,
        pltpu.SemaphoreType.DMA,
        pltpu.SemaphoreType.DMA,
        pltpu.SemaphoreType.DMA,
        pltpu.SemaphoreType.DMA,
    ],
    compiler_params=pltpu.CompilerParams(
        use_tc_tiling_on_sc=True, needs_layout_passes=False),
)
def _emb_stream(idx_hbm, table_t_hbm, out_hbm, idx_all, strip_0, strip_1,
                strip_2, strip_3, strip_4, strip_5, hit_r, hit_i, sr_s, si_s,
                hist, bases, nxt, outbuf, ssem_0, ssem_1, ssem_2, ssem_3,
                ssem_4, ssem_5, osem):
    wid = lax.axis_index("s") * _NC + lax.axis_index("c")
    lo_strip = wid * STRIPS_BASE + jnp.minimum(wid, STRIPS_REM)
    n_strips = STRIPS_BASE + jnp.where(wid < STRIPS_REM, 1, 0)
    lo_c = lo_strip
    hi_c = lo_strip + n_strips

    pltpu.sync_copy(idx_hbm, idx_all)

    strips = (strip_0, strip_1, strip_2, strip_3, strip_4, strip_5)
    ssems = (ssem_0, ssem_1, ssem_2, ssem_3, ssem_4, ssem_5)
    NBUF = 6

    def fetch_strip(g, m):
        col0 = pl.multiple_of((lo_strip + g) * 128, 128)
        return pltpu.async_copy(
            table_t_hbm.at[:, pl.ds(col0, 128)], strips[m], ssems[m])

    for m in range(NBUF):
        fetch_strip(m, m)

    iota = lax.iota(jnp.int32, LANES)
    ones = jnp.full((LANES,), 1, jnp.int32)
    zeros = jnp.full((LANES,), 0, jnp.int32)
    lane0 = iota == zeros

    def scan(n, nh):
        v = idx_all[pl.ds(n * LANES, LANES)]
        c = lax.shift_right_logical(v, 7)
        m = (c >= _splat(lo_c)) & (c < _splat(hi_c))
        cnt = plsc.cumsum(jnp.where(m, ones, zeros))
        dest = _splat(nh) + cnt - 1
        plsc.store_scatter(hit_r, [dest], v, mask=m)
        pos = _splat(n * LANES) + iota
        plsc.store_scatter(hit_i, [dest], pos, mask=m)
        return nh + cnt[LANES - 1]
    nh = lax.fori_loop(0, N_IDX_CHUNKS, scan, jnp.int32(0), unroll=4)

    def clear(n, _):
        hist[pl.ds(n * LANES, LANES)] = zeros
        return ()
    lax.fori_loop(0, NBUCKET // LANES, clear, (), unroll=False)

    def count(j, _):
        rv = hit_r[pl.ds(j, LANES)]
        strip = lax.shift_right_logical(rv[0], 7) - lo_strip
        ch = lax.shift_right_logical(strip, 4)
        lp = lax.rem(strip, jnp.int32(LANES))
        hv = hist[pl.ds(ch * LANES, LANES)]
        hist[pl.ds(ch * LANES, LANES)] = hv + jnp.where(
            iota == _splat(lp), ones, zeros)
        return ()
    lax.fori_loop(0, nh, count, (), unroll=False)

    def prefix(n, carry):
        hv = hist[pl.ds(n * LANES, LANES)]
        cs = plsc.cumsum(hv)
        bases[pl.ds(n * LANES, LANES)] = _splat(carry) + cs - hv
        nxt[pl.ds(n * LANES, LANES)] = _splat(carry) + cs - hv
        return carry + cs[LANES - 1]
    lax.fori_loop(0, NBUCKET // LANES, prefix, jnp.int32(0), unroll=False)

    def place(j, _):
        rv = hit_r[pl.ds(j, LANES)]
        iv = hit_i[pl.ds(j, LANES)]
        r = rv[0]
        i_out = iv[0]
        strip = lax.shift_right_logical(r, 7) - lo_strip
        sv = _splat(strip)
        n0 = plsc.load_gather(nxt, [sv])[0]
        nv0 = _splat(n0)
        plsc.store_scatter(sr_s, [nv0], _splat(r), mask=lane0)
        plsc.store_scatter(si_s, [nv0], _splat(i_out), mask=lane0)
        plsc.store_scatter(nxt, [sv], nv0 + 1, mask=lane0)
        return ()
    lax.fori_loop(0, nh, place, (), unroll=False)

    def extract_hits(k0, k1, buf):
        def one(k, _):
            rv = sr_s[pl.ds(k, LANES)]
            iv = si_s[pl.ds(k, LANES)]
            r = rv[0]
            i_out = iv[0]
            lane = lax.rem(r, jnp.int32(128))
            colv = _splat(lane)
            slot = lax.rem(k, jnp.int32(SLOTS))
            sbase = slot * SLOT_STRIDE
            for t in range(7):
                rowv = iota + t * LANES
                if t == 6:
                    rowv = jnp.minimum(rowv, DIM - 1)
                g16 = plsc.load_gather(buf, [rowv, colv])
                outbuf[pl.ds(sbase + t * LANES, LANES)] = g16

            @pl.when(k >= SLOTS)
            def _():
                pltpu.make_async_copy(
                    out_hbm.at[pl.ds(0, ROW_PAD)],
                    outbuf.at[pl.ds(sbase, ROW_PAD)],
                    osem,
                ).wait()

            pltpu.async_copy(
                outbuf.at[pl.ds(sbase, ROW_PAD)],
                out_hbm.at[pl.ds(i_out * ROW_PAD, ROW_PAD)],
                osem,
            )
            return ()
        lax.fori_loop(k0, k1, one, (), unroll=False)

    def strip_body(g, _):
        gv = _splat(g)
        start = plsc.load_gather(bases, [gv])[0]
        cnt_g = plsc.load_gather(hist, [gv])[0]
        end = start + cnt_g

        mod = lax.rem(g, jnp.int32(NBUF))
        for m in range(NBUF):
            @pl.when(mod == m)
            def _(m=m):
                pltpu.make_async_copy(
                    table_t_hbm.at[:, pl.ds(0, 128)],
                    strips[m], ssems[m]).wait()
                extract_hits(start, end, strips[m])

                @pl.when(g + NBUF < n_strips)
                def _():
                    fetch_strip(g + NBUF, m)

        return ()

    lax.fori_loop(0, n_strips, strip_body, (), unroll=False)

    def final_drain(_, __):
        pltpu.make_async_copy(
            out_hbm.at[pl.ds(0, ROW_PAD)],
            outbuf.at[pl.ds(0, ROW_PAD)],
            osem,
        ).wait()
        return ()
    lax.fori_loop(0, jnp.minimum(nh, SLOTS), final_drain, (), unroll=False)


def kernel(inputs, table):
    idx = inputs.reshape(BATCH)
    out_pad = _emb_stream(idx, table.T)
    return out_pad.reshape(BATCH, ROW_PAD)[:, :DIM]


# speedup vs baseline: 1.8599x; 1.0655x over previous
import functools

import jax
import jax.numpy as jnp
from jax import lax
from jax.experimental import pallas as pl
from jax.experimental.pallas import tpu as pltpu
from jax.experimental.pallas import tpu_sc as plsc

BATCH = 16384
DIM = 100
ROW_PAD = 104

_INFO = plsc.get_sparse_core_info()
_NC = _INFO.num_cores
_NS = _INFO.num_subcores
NW = _NC * _NS
LANES = 16

VOCAB_PAD = 1000064
N_TILE_COLS = VOCAB_PAD // 128
STRIPS_BASE = N_TILE_COLS // NW
STRIPS_REM = N_TILE_COLS % NW

HIT_CAP = 1024
N_IDX_CHUNKS = BATCH // LANES
SLOTS = 16
SLOT_STRIDE = 112
NBUCKET = 272


def _splat(x):
    return jnp.full((LANES,), x, jnp.int32)


@functools.partial(
    pl.kernel,
    mesh=plsc.VectorSubcoreMesh(core_axis_name="c", subcore_axis_name="s"),
    out_type=jax.ShapeDtypeStruct((BATCH * ROW_PAD,), jnp.float32),
    scratch_types=[
        pltpu.VMEM((BATCH,), jnp.int32),
        pltpu.VMEM((100, 128), jnp.float32),
        pltpu.VMEM((100, 128), jnp.float32),
        pltpu.VMEM((100, 128), jnp.float32),
        pltpu.VMEM((100, 128), jnp.float32),
        pltpu.VMEM((100, 128), jnp.float32),
        pltpu.VMEM((100, 128), jnp.float32),
        pltpu.VMEM((HIT_CAP + LANES,), jnp.int32),
        pltpu.VMEM((HIT_CAP + LANES,), jnp.int32),
        pltpu.VMEM((HIT_CAP + LANES,), jnp.int32),
        pltpu.VMEM((HIT_CAP + LANES,), jnp.int32),
        pltpu.VMEM((NBUCKET,), jnp.int32),
        pltpu.VMEM((NBUCKET,), jnp.int32),
        pltpu.VMEM((NBUCKET,), jnp.int32),
        pltpu.VMEM((NBUCKET,), jnp.int32),
        pltpu.VMEM((SLOTS * SLOT_STRIDE + LANES,), jnp.float32),
        pltpu.SemaphoreType.DMA,
        pltpu.SemaphoreType.DMA,
        pltpu.SemaphoreType.DMA,
        pltpu.SemaphoreType.DMA,
        pltpu.SemaphoreType.DMA,
        pltpu.SemaphoreType.DMA,
        pltpu.SemaphoreType.DMA,
    ],
    compiler_params=pltpu.CompilerParams(
        use_tc_tiling_on_sc=True, needs_layout_passes=False),
)
def _emb_stream(idx_hbm, table_t_hbm, out_hbm, idx_all, strip_0, strip_1,
                strip_2, strip_3, strip_4, strip_5, hit_r, hit_i, sr_s, si_s,
                hist, bases, nxt, occ, outbuf, ssem_0, ssem_1, ssem_2, ssem_3,
                ssem_4, ssem_5, osem):
    wid = lax.axis_index("s") * _NC + lax.axis_index("c")
    lo_strip = wid * STRIPS_BASE + jnp.minimum(wid, STRIPS_REM)
    n_strips = STRIPS_BASE + jnp.where(wid < STRIPS_REM, 1, 0)
    lo_c = lo_strip
    hi_c = lo_strip + n_strips

    pltpu.sync_copy(idx_hbm, idx_all)

    strips = (strip_0, strip_1, strip_2, strip_3, strip_4, strip_5)
    ssems = (ssem_0, ssem_1, ssem_2, ssem_3, ssem_4, ssem_5)
    NBUF = 6

    def fetch_strip(sid, m):
        col0 = pl.multiple_of((lo_strip + sid) * 128, 128)
        return pltpu.async_copy(
            table_t_hbm.at[:, pl.ds(col0, 128)], strips[m], ssems[m])

    iota = lax.iota(jnp.int32, LANES)
    ones = jnp.full((LANES,), 1, jnp.int32)
    zeros = jnp.full((LANES,), 0, jnp.int32)
    lane0 = iota == zeros

    def scan(n, nh):
        v = idx_all[pl.ds(n * LANES, LANES)]
        c = lax.shift_right_logical(v, 7)
        m = (c >= _splat(lo_c)) & (c < _splat(hi_c))
        cnt = plsc.cumsum(jnp.where(m, ones, zeros))
        dest = _splat(nh) + cnt - 1
        plsc.store_scatter(hit_r, [dest], v, mask=m)
        pos = _splat(n * LANES) + iota
        plsc.store_scatter(hit_i, [dest], pos, mask=m)
        return nh + cnt[LANES - 1]
    nh = lax.fori_loop(0, N_IDX_CHUNKS, scan, jnp.int32(0), unroll=4)

    def clear(n, _):
        hist[pl.ds(n * LANES, LANES)] = zeros
        return ()
    lax.fori_loop(0, NBUCKET // LANES, clear, (), unroll=False)

    def count(j, _):
        rv = hit_r[pl.ds(j, LANES)]
        strip = lax.shift_right_logical(rv[0], 7) - lo_strip
        ch = lax.shift_right_logical(strip, 4)
        lp = lax.rem(strip, jnp.int32(LANES))
        hv = hist[pl.ds(ch * LANES, LANES)]
        hist[pl.ds(ch * LANES, LANES)] = hv + jnp.where(
            iota == _splat(lp), ones, zeros)
        return ()
    lax.fori_loop(0, nh, count, (), unroll=False)

    def prefix(n, carry):
        hv = hist[pl.ds(n * LANES, LANES)]
        cs = plsc.cumsum(hv)
        bases[pl.ds(n * LANES, LANES)] = _splat(carry) + cs - hv
        nxt[pl.ds(n * LANES, LANES)] = _splat(carry) + cs - hv
        return carry + cs[LANES - 1]
    lax.fori_loop(0, NBUCKET // LANES, prefix, jnp.int32(0), unroll=False)

    def place(j, _):
        rv = hit_r[pl.ds(j, LANES)]
        iv = hit_i[pl.ds(j, LANES)]
        r = rv[0]
        i_out = iv[0]
        strip = lax.shift_right_logical(r, 7) - lo_strip
        sv = _splat(strip)
        n0 = plsc.load_gather(nxt, [sv])[0]
        nv0 = _splat(n0)
        plsc.store_scatter(sr_s, [nv0], _splat(r), mask=lane0)
        plsc.store_scatter(si_s, [nv0], _splat(i_out), mask=lane0)
        plsc.store_scatter(nxt, [sv], nv0 + 1, mask=lane0)
        return ()
    lax.fori_loop(0, nh, place, (), unroll=False)

    def build_occ(n, n_occ):
        hv = hist[pl.ds(n * LANES, LANES)]
        m = hv > zeros
        cnt = plsc.cumsum(jnp.where(m, ones, zeros))
        dest = _splat(n_occ) + cnt - 1
        plsc.store_scatter(occ, [dest], _splat(n * LANES) + iota, mask=m)
        return n_occ + cnt[LANES - 1]
    n_occ = lax.fori_loop(0, NBUCKET // LANES, build_occ, jnp.int32(0),
                          unroll=False)

    for m in range(NBUF):
        @pl.when(m < n_occ)
        def _(m=m):
            sid0 = plsc.load_gather(occ, [_splat(m)])[0]
            fetch_strip(sid0, m)

    def extract_hits(k0, k1, buf):
        def one(k, _):
            rv = sr_s[pl.ds(k, LANES)]
            iv = si_s[pl.ds(k, LANES)]
            r = rv[0]
            i_out = iv[0]
            lane = lax.rem(r, jnp.int32(128))
            colv = _splat(lane)
            slot = lax.rem(k, jnp.int32(SLOTS))
            sbase = slot * SLOT_STRIDE
            for t in range(7):
                rowv = iota + t * LANES
                if t == 6:
                    rowv = jnp.minimum(rowv, DIM - 1)
                g16 = plsc.load_gather(buf, [rowv, colv])
                outbuf[pl.ds(sbase + t * LANES, LANES)] = g16

            @pl.when(k >= SLOTS)
            def _():
                pltpu.make_async_copy(
                    out_hbm.at[pl.ds(0, ROW_PAD)],
                    outbuf.at[pl.ds(sbase, ROW_PAD)],
                    osem,
                ).wait()

            pltpu.async_copy(
                outbuf.at[pl.ds(sbase, ROW_PAD)],
                out_hbm.at[pl.ds(i_out * ROW_PAD, ROW_PAD)],
                osem,
            )
            return ()
        lax.fori_loop(k0, k1, one, (), unroll=False)

    def strip_body(g, _):
        gv = _splat(g)
        sid = plsc.load_gather(occ, [gv])[0]
        sv = _splat(sid)
        start = plsc.load_gather(bases, [sv])[0]
        cnt_g = plsc.load_gather(hist, [sv])[0]
        end = start + cnt_g

        mod = lax.rem(g, jnp.int32(NBUF))
        for m in range(NBUF):
            @pl.when(mod == m)
            def _(m=m):
                pltpu.make_async_copy(
                    table_t_hbm.at[:, pl.ds(0, 128)],
                    strips[m], ssems[m]).wait()
                extract_hits(start, end, strips[m])

                @pl.when(g + NBUF < n_occ)
                def _():
                    sid_n = plsc.load_gather(occ, [_splat(g) + NBUF])[0]
                    fetch_strip(sid_n, m)

        return ()

    lax.fori_loop(0, n_occ, strip_body, (), unroll=False)

    def final_drain(_, __):
        pltpu.make_async_copy(
            out_hbm.at[pl.ds(0, ROW_PAD)],
            outbuf.at[pl.ds(0, ROW_PAD)],
            osem,
        ).wait()
        return ()
    lax.fori_loop(0, jnp.minimum(nh, SLOTS), final_drain, (), unroll=False)


def kernel(inputs, table):
    idx = inputs.reshape(BATCH)
    out_pad = _emb_stream(idx, table.T)
    return out_pad.reshape(BATCH, ROW_PAD)[:, :DIM]
